# speedup vs baseline: 35.8097x; 35.8097x over previous
import functools

import jax
import jax.numpy as jnp
from jax import lax
from jax.experimental import pallas as pl
from jax.experimental.pallas import tpu as pltpu
from jax.experimental.pallas import tpu_sc as plsc

E = 8
D_IN = 2048
Z_DIM = 2048
B = 128

ROWS = E * Z_DIM
ROW_BLK = 2048


def _matvec_body(w_ref, z_ref, b_ref, y_ref):
    y_ref[...] = (
        jnp.dot(w_ref[...], z_ref[...], preferred_element_type=jnp.float32)
        + b_ref[...]
    )


_matvec = pl.pallas_call(
    _matvec_body,
    grid=(ROWS // ROW_BLK,),
    in_specs=[
        pl.BlockSpec((ROW_BLK, D_IN), lambda i: (i, 0)),
        pl.BlockSpec((D_IN, 1), lambda i: (0, 0)),
        pl.BlockSpec((ROW_BLK, 1), lambda i: (i, 0)),
    ],
    out_specs=pl.BlockSpec((ROW_BLK, 1), lambda i: (i, 0)),
    out_shape=jax.ShapeDtypeStruct((ROWS, 1), jnp.float32),
)


_NW_USED = 16
_B_PER_W = B // _NW_USED

_sc_mesh = plsc.VectorSubcoreMesh(core_axis_name="c", subcore_axis_name="s")


@functools.partial(
    pl.kernel,
    out_type=jax.ShapeDtypeStruct((B, Z_DIM), jnp.float32),
    mesh=_sc_mesh,
    scratch_types=[
        pltpu.VMEM((_B_PER_W,), jnp.int32),
        pltpu.VMEM((_B_PER_W, Z_DIM), jnp.float32),
        pltpu.SemaphoreType.DMA,
    ],
)
def _sc_gather(y_hbm, idx_hbm, out_hbm, idx_v, rows_v, sem):
    wid = lax.axis_index("s") * 2 + lax.axis_index("c")

    @pl.when(wid < _NW_USED)
    def _():
        base = wid * _B_PER_W
        pltpu.sync_copy(idx_hbm.at[pl.ds(base, _B_PER_W)], idx_v)
        pltpu.async_copy(y_hbm.at[idx_v], rows_v, sem).wait()
        pltpu.sync_copy(rows_v, out_hbm.at[pl.ds(base, _B_PER_W)])


def kernel(indices, z, W, b):
    idx = indices.astype(jnp.int32)
    w_flat = W.reshape(ROWS, D_IN)
    b_flat = b.reshape(ROWS, 1)
    z_col = z.reshape(D_IN, 1)
    y = _matvec(w_flat, z_col, b_flat).reshape(E, Z_DIM)
    return _sc_gather(y, idx)


# speedup vs baseline: 36.8060x; 1.0278x over previous
import functools

import jax
import jax.numpy as jnp
from jax import lax
from jax.experimental import pallas as pl
from jax.experimental.pallas import tpu as pltpu
from jax.experimental.pallas import tpu_sc as plsc

E = 8
D_IN = 2048
Z_DIM = 2048
B = 128

ROWS = E * Z_DIM
ROW_BLK = 1024


def _matvec_body(w_ref, z_ref, b_ref, y_ref):
    y_ref[...] = (
        jnp.dot(w_ref[...], z_ref[...], preferred_element_type=jnp.float32)
        + b_ref[...]
    )


_matvec = pl.pallas_call(
    _matvec_body,
    grid=(ROWS // ROW_BLK,),
    in_specs=[
        pl.BlockSpec((ROW_BLK, D_IN), lambda i: (i, 0)),
        pl.BlockSpec((D_IN, 1), lambda i: (0, 0)),
        pl.BlockSpec((ROW_BLK, 1), lambda i: (i, 0)),
    ],
    out_specs=pl.BlockSpec((ROW_BLK, 1), lambda i: (i, 0)),
    out_shape=jax.ShapeDtypeStruct((ROWS, 1), jnp.float32),
)


_NW_USED = 16
_B_PER_W = B // _NW_USED

_sc_mesh = plsc.VectorSubcoreMesh(core_axis_name="c", subcore_axis_name="s")


@functools.partial(
    pl.kernel,
    out_type=jax.ShapeDtypeStruct((B, Z_DIM), jnp.float32),
    mesh=_sc_mesh,
    scratch_types=[
        pltpu.VMEM((_B_PER_W,), jnp.int32),
        pltpu.VMEM((_B_PER_W, Z_DIM), jnp.float32),
        pltpu.SemaphoreType.DMA,
    ],
)
def _sc_gather(y_hbm, idx_hbm, out_hbm, idx_v, rows_v, sem):
    wid = lax.axis_index("s") * 2 + lax.axis_index("c")

    @pl.when(wid < _NW_USED)
    def _():
        base = wid * _B_PER_W
        pltpu.sync_copy(idx_hbm.at[pl.ds(base, _B_PER_W)], idx_v)
        pltpu.async_copy(y_hbm.at[idx_v], rows_v, sem).wait()
        pltpu.sync_copy(rows_v, out_hbm.at[pl.ds(base, _B_PER_W)])


def kernel(indices, z, W, b):
    idx = indices.astype(jnp.int32)
    w_flat = W.reshape(ROWS, D_IN)
    b_flat = b.reshape(ROWS, 1)
    z_col = z.reshape(D_IN, 1)
    y = _matvec(w_flat, z_col, b_flat).reshape(E, Z_DIM)
    return _sc_gather(y, idx)
